# baseline (speedup 1.0000x reference)
import functools

import jax
import jax.numpy as jnp
from jax import lax
from jax.experimental import pallas as pl
from jax.experimental.pallas import tpu as pltpu
from jax.experimental.pallas import tpu_sc as plsc

BATCH = 2048
IN_FEATURES = 2048
OUT_FEATURES = 2048
WPO = 16

_NC = 2
_NS = 16
_NW = _NC * _NS
_O_PER_W = OUT_FEATURES // _NW
_CHUNK_ROWS = 16
_N_CHUNKS = _O_PER_W // _CHUNK_ROWS


def _densify_sc(connections, weights):
    mesh = plsc.VectorSubcoreMesh(core_axis_name="c", subcore_axis_name="s")

    @functools.partial(
        pl.kernel,
        mesh=mesh,
        out_type=jax.ShapeDtypeStruct((OUT_FEATURES, IN_FEATURES), jnp.float32),
        scratch_types=[
            pltpu.VMEM((_O_PER_W, WPO), jnp.int32),
            pltpu.VMEM((_O_PER_W, WPO), jnp.float32),
            pltpu.VMEM((_CHUNK_ROWS, IN_FEATURES), jnp.float32),
            pltpu.VMEM((_CHUNK_ROWS, IN_FEATURES), jnp.float32),
            pltpu.SemaphoreType.DMA,
            pltpu.SemaphoreType.DMA,
        ],
        compiler_params=pltpu.CompilerParams(needs_layout_passes=False),
    )
    def densify(conn_hbm, wt_hbm, wd_hbm, conn_v, wt_v, buf0, buf1, sem0, sem1):
        wid = lax.axis_index("s") * _NC + lax.axis_index("c")
        base = wid * _O_PER_W
        pltpu.sync_copy(conn_hbm.at[pl.ds(base, _O_PER_W), :], conn_v)
        pltpu.sync_copy(wt_hbm.at[pl.ds(base, _O_PER_W), :], wt_v)
        lane = lax.iota(jnp.int32, 16)
        zeros16 = jnp.zeros((16,), jnp.float32)
        bufs = (buf0, buf1)
        sems = (sem0, sem1)

        for buf in bufs:
            def _zero(j, _):
                for r in range(_CHUNK_ROWS):
                    buf[r, pl.ds(j * 16, 16)] = zeros16
                return 0
            lax.fori_loop(0, IN_FEATURES // 16, _zero, 0)

        def taps(c, w):
            ridx = c * _CHUNK_ROWS + lane
            widx = jnp.full((16,), w, jnp.int32)
            cols = plsc.load_gather(conn_v, [ridx, widx])
            vals = plsc.load_gather(wt_v, [ridx, widx])
            return cols, vals

        copies = [None, None]
        for c in range(_N_CHUNKS):
            slot = c % 2
            buf = bufs[slot]
            if copies[slot] is not None:
                copies[slot].wait()
                for w in range(WPO):
                    cols_prev, _ = taps(c - 2, w)
                    plsc.store_scatter(buf, [lane, cols_prev], zeros16)
            for w in range(WPO):
                cols, vals = taps(c, w)
                plsc.addupdate_scatter(buf, [lane, cols], vals)
            row0 = base + c * _CHUNK_ROWS
            copies[slot] = pltpu.async_copy(
                buf, wd_hbm.at[pl.ds(row0, _CHUNK_ROWS), :], sems[slot])
        for cp in copies:
            if cp is not None:
                cp.wait()

    return densify(connections, weights)


def _matmul_tc(x, wd):
    bo = 1024

    def mm(x_ref, w_ref, o_ref):
        wb = w_ref[...].astype(jnp.bfloat16)
        o_ref[...] = lax.dot_general(
            x_ref[...], wb, (((1,), (1,)), ((), ())),
            preferred_element_type=jnp.float32)

    return pl.pallas_call(
        mm,
        grid=(OUT_FEATURES // bo,),
        in_specs=[
            pl.BlockSpec((BATCH, IN_FEATURES), lambda o: (0, 0)),
            pl.BlockSpec((bo, IN_FEATURES), lambda o: (o, 0)),
        ],
        out_specs=pl.BlockSpec((BATCH, bo), lambda o: (0, o)),
        out_shape=jax.ShapeDtypeStruct((BATCH, OUT_FEATURES), jnp.float32),
    )(x, wd)


def kernel(input, connections, weights):
    x_bf = input.astype(jnp.bfloat16)
    wd = _densify_sc(connections, weights)
    return _matmul_tc(x_bf, wd)


# speedup vs baseline: 1.0350x; 1.0350x over previous
import functools

import jax
import jax.numpy as jnp
from jax import lax
from jax.experimental import pallas as pl
from jax.experimental.pallas import tpu as pltpu
from jax.experimental.pallas import tpu_sc as plsc

BATCH = 2048
IN_FEATURES = 2048
OUT_FEATURES = 2048
WPO = 16

_NC = 2
_NS = 16
_NW = _NC * _NS
_O_PER_W = OUT_FEATURES // _NW
_CHUNK_ROWS = 16
_N_CHUNKS = _O_PER_W // _CHUNK_ROWS


def _densify_sc(connections, weights):
    mesh = plsc.VectorSubcoreMesh(core_axis_name="c", subcore_axis_name="s")

    @functools.partial(
        pl.kernel,
        mesh=mesh,
        out_type=jax.ShapeDtypeStruct((OUT_FEATURES, IN_FEATURES), jnp.float32),
        scratch_types=[
            pltpu.VMEM((_O_PER_W, WPO), jnp.int32),
            pltpu.VMEM((_O_PER_W, WPO), jnp.float32),
            pltpu.VMEM((_CHUNK_ROWS, IN_FEATURES), jnp.float32),
            pltpu.VMEM((_CHUNK_ROWS, IN_FEATURES), jnp.float32),
            pltpu.SemaphoreType.DMA,
            pltpu.SemaphoreType.DMA,
        ],
        compiler_params=pltpu.CompilerParams(needs_layout_passes=False),
    )
    def densify(conn_hbm, wt_hbm, wd_hbm, conn_v, wt_v, buf0, buf1, sem0, sem1):
        wid = lax.axis_index("s") * _NC + lax.axis_index("c")
        base = wid * _O_PER_W
        pltpu.sync_copy(conn_hbm.at[pl.ds(base, _O_PER_W), :], conn_v)
        pltpu.sync_copy(wt_hbm.at[pl.ds(base, _O_PER_W), :], wt_v)
        lane = lax.iota(jnp.int32, 16)
        zeros16 = jnp.zeros((16,), jnp.float32)
        bufs = (buf0, buf1)
        sems = (sem0, sem1)

        for buf in bufs:
            def _zero(j, _):
                for r in range(_CHUNK_ROWS):
                    buf[r, pl.ds(j * 16, 16)] = zeros16
                return 0
            lax.fori_loop(0, IN_FEATURES // 16, _zero, 0)

        def taps(c, w):
            ridx = c * _CHUNK_ROWS + lane
            widx = jnp.full((16,), w, jnp.int32)
            cols = plsc.load_gather(conn_v, [ridx, widx])
            vals = plsc.load_gather(wt_v, [ridx, widx])
            return cols, vals

        copies = [None, None]
        for c in range(_N_CHUNKS):
            slot = c % 2
            buf = bufs[slot]
            if copies[slot] is not None:
                copies[slot].wait()
                for w in range(WPO):
                    cols_prev, _ = taps(c - 2, w)
                    plsc.store_scatter(buf, [lane, cols_prev], zeros16)
            for w in range(WPO):
                cols, vals = taps(c, w)
                plsc.addupdate_scatter(buf, [lane, cols], vals)
            row0 = base + c * _CHUNK_ROWS
            copies[slot] = pltpu.async_copy(
                buf, wd_hbm.at[pl.ds(row0, _CHUNK_ROWS), :], sems[slot])
        for cp in copies:
            if cp is not None:
                cp.wait()

    return densify(connections, weights)


def _matmul_tc(x, wd):
    bo = 512

    def mm(x_ref, w_ref, o_ref, xb_ref):
        @pl.when(pl.program_id(0) == 0)
        def _():
            xb_ref[...] = x_ref[...].astype(jnp.bfloat16)
        wb = w_ref[...].astype(jnp.bfloat16)
        o_ref[...] = lax.dot_general(
            xb_ref[...], wb, (((1,), (1,)), ((), ())),
            preferred_element_type=jnp.float32)

    return pl.pallas_call(
        mm,
        grid=(OUT_FEATURES // bo,),
        in_specs=[
            pl.BlockSpec((BATCH, IN_FEATURES), lambda o: (0, 0)),
            pl.BlockSpec((bo, IN_FEATURES), lambda o: (o, 0)),
        ],
        out_specs=pl.BlockSpec((BATCH, bo), lambda o: (0, o)),
        out_shape=jax.ShapeDtypeStruct((BATCH, OUT_FEATURES), jnp.float32),
        scratch_shapes=[pltpu.VMEM((BATCH, IN_FEATURES), jnp.bfloat16)],
    )(x, wd)


def kernel(input, connections, weights):
    wd = _densify_sc(connections, weights)
    return _matmul_tc(input, wd)


# speedup vs baseline: 1.0697x; 1.0335x over previous
import functools

import jax
import jax.numpy as jnp
from jax import lax
from jax.experimental import pallas as pl
from jax.experimental.pallas import tpu as pltpu
from jax.experimental.pallas import tpu_sc as plsc

BATCH = 2048
IN_FEATURES = 2048
OUT_FEATURES = 2048
WPO = 16

_NC = 2
_NS = 16
_NW = _NC * _NS
_O_PER_W = OUT_FEATURES // _NW
_CHUNK_ROWS = 16
_N_CHUNKS = _O_PER_W // _CHUNK_ROWS


def _densify_sc(cw):
    mesh = plsc.VectorSubcoreMesh(core_axis_name="c", subcore_axis_name="s")

    @functools.partial(
        pl.kernel,
        mesh=mesh,
        out_type=jax.ShapeDtypeStruct((OUT_FEATURES, IN_FEATURES), jnp.float32),
        scratch_types=[
            pltpu.VMEM((_O_PER_W, 2 * WPO), jnp.int32),
            pltpu.VMEM((_CHUNK_ROWS, IN_FEATURES), jnp.float32),
            pltpu.VMEM((_CHUNK_ROWS, IN_FEATURES), jnp.float32),
            pltpu.SemaphoreType.DMA,
            pltpu.SemaphoreType.DMA,
        ],
        compiler_params=pltpu.CompilerParams(needs_layout_passes=False),
    )
    def densify(cw_hbm, wd_hbm, cw_v, buf0, buf1, sem0, sem1):
        wid = lax.axis_index("s") * _NC + lax.axis_index("c")
        base = wid * _O_PER_W
        pltpu.sync_copy(cw_hbm.at[pl.ds(base, _O_PER_W), :], cw_v)
        lane = lax.iota(jnp.int32, 16)
        zeros16 = jnp.zeros((16,), jnp.float32)
        bufs = (buf0, buf1)
        sems = (sem0, sem1)

        for buf in bufs:
            def _zero(j, _):
                for r in range(_CHUNK_ROWS):
                    buf[r, pl.ds(j * 16, 16)] = zeros16
                return 0
            lax.fori_loop(0, IN_FEATURES // 16, _zero, 0)

        def taps(c, w):
            ridx = c * _CHUNK_ROWS + lane
            widx = jnp.full((16,), w, jnp.int32)
            cols = plsc.load_gather(cw_v, [ridx, widx])
            vals = plsc.bitcast(
                plsc.load_gather(cw_v, [ridx, widx + WPO]), jnp.float32)
            return cols, vals

        copies = [None, None]
        for c in range(_N_CHUNKS):
            slot = c % 2
            buf = bufs[slot]
            if copies[slot] is not None:
                copies[slot].wait()
                for w in range(WPO):
                    cols_prev, _ = taps(c - 2, w)
                    plsc.store_scatter(buf, [lane, cols_prev], zeros16)
            for w in range(WPO):
                cols, vals = taps(c, w)
                plsc.addupdate_scatter(buf, [lane, cols], vals)
            row0 = base + c * _CHUNK_ROWS
            copies[slot] = pltpu.async_copy(
                buf, wd_hbm.at[pl.ds(row0, _CHUNK_ROWS), :], sems[slot])
        for cp in copies:
            if cp is not None:
                cp.wait()

    return densify(cw)


def _matmul_tc(x, wd):
    bo = 512

    def mm(x_ref, w_ref, o_ref, xb_ref):
        @pl.when(pl.program_id(0) == 0)
        def _():
            xb_ref[...] = x_ref[...].astype(jnp.bfloat16)
        wb = w_ref[...].astype(jnp.bfloat16)
        o_ref[...] = lax.dot_general(
            xb_ref[...], wb, (((1,), (1,)), ((), ())),
            preferred_element_type=jnp.float32)

    return pl.pallas_call(
        mm,
        grid=(OUT_FEATURES // bo,),
        in_specs=[
            pl.BlockSpec((BATCH, IN_FEATURES), lambda o: (0, 0)),
            pl.BlockSpec((bo, IN_FEATURES), lambda o: (o, 0)),
        ],
        out_specs=pl.BlockSpec((BATCH, bo), lambda o: (0, o)),
        out_shape=jax.ShapeDtypeStruct((BATCH, OUT_FEATURES), jnp.float32),
        scratch_shapes=[pltpu.VMEM((BATCH, IN_FEATURES), jnp.bfloat16)],
    )(x, wd)


def kernel(input, connections, weights):
    cw = jnp.concatenate(
        [connections, lax.bitcast_convert_type(weights, jnp.int32)], axis=1)
    wd = _densify_sc(cw)
    return _matmul_tc(input, wd)


# speedup vs baseline: 1.0857x; 1.0150x over previous
import functools

import jax
import jax.numpy as jnp
from jax import lax
from jax.experimental import pallas as pl
from jax.experimental.pallas import tpu as pltpu
from jax.experimental.pallas import tpu_sc as plsc

BATCH = 2048
IN_FEATURES = 2048
OUT_FEATURES = 2048
WPO = 16

_NC = 2
_NS = 16
_NW = _NC * _NS
_O_PER_W = OUT_FEATURES // _NW
_CHUNK_ROWS = 16
_N_CHUNKS = _O_PER_W // _CHUNK_ROWS


def _densify_sc(cw):
    mesh = plsc.VectorSubcoreMesh(core_axis_name="c", subcore_axis_name="s")

    @functools.partial(
        pl.kernel,
        mesh=mesh,
        out_type=jax.ShapeDtypeStruct((OUT_FEATURES, IN_FEATURES), jnp.float32),
        scratch_types=[
            pltpu.VMEM((_O_PER_W, 2 * WPO), jnp.int32),
            pltpu.VMEM((_CHUNK_ROWS, IN_FEATURES), jnp.float32),
            pltpu.VMEM((_CHUNK_ROWS, IN_FEATURES), jnp.float32),
            pltpu.SemaphoreType.DMA,
            pltpu.SemaphoreType.DMA,
        ],
        compiler_params=pltpu.CompilerParams(needs_layout_passes=False),
    )
    def densify(cw_hbm, wd_hbm, cw_v, buf0, buf1, sem0, sem1):
        wid = lax.axis_index("s") * _NC + lax.axis_index("c")
        base = wid * _O_PER_W
        pltpu.sync_copy(cw_hbm.at[pl.ds(base, _O_PER_W), :], cw_v)
        lane = lax.iota(jnp.int32, 16)
        zeros16 = jnp.zeros((16,), jnp.float32)
        bufs = (buf0, buf1)
        sems = (sem0, sem1)

        for buf in bufs:
            def _zero(j, _):
                for r in range(_CHUNK_ROWS):
                    buf[r, pl.ds(j * 16, 16)] = zeros16
                return 0
            lax.fori_loop(0, IN_FEATURES // 16, _zero, 0)

        def taps(c, w):
            ridx = c * _CHUNK_ROWS + lane
            widx = jnp.full((16,), w, jnp.int32)
            cols = plsc.load_gather(cw_v, [ridx, widx])
            vals = plsc.bitcast(
                plsc.load_gather(cw_v, [ridx, widx + WPO]), jnp.float32)
            return cols, vals

        copies = [None, None]
        for c in range(_N_CHUNKS):
            slot = c % 2
            buf = bufs[slot]
            if copies[slot] is not None:
                copies[slot].wait()
                for w in range(WPO):
                    cols_prev, _ = taps(c - 2, w)
                    plsc.store_scatter(buf, [lane, cols_prev], zeros16)
            for w in range(WPO):
                cols, vals = taps(c, w)
                plsc.addupdate_scatter(buf, [lane, cols], vals)
            row0 = base + c * _CHUNK_ROWS
            copies[slot] = pltpu.async_copy(
                buf, wd_hbm.at[pl.ds(row0, _CHUNK_ROWS), :], sems[slot])
        for cp in copies:
            if cp is not None:
                cp.wait()

    return densify(cw)


def _matmul_tc(x, wd):
    bo = 512

    def mm(x_ref, w_ref, o_ref):
        wb = w_ref[...].astype(jnp.bfloat16)
        o_ref[...] = lax.dot_general(
            x_ref[...], wb, (((1,), (1,)), ((), ())),
            preferred_element_type=jnp.float32)

    return pl.pallas_call(
        mm,
        grid=(OUT_FEATURES // bo,),
        in_specs=[
            pl.BlockSpec((BATCH, IN_FEATURES), lambda o: (0, 0)),
            pl.BlockSpec((bo, IN_FEATURES), lambda o: (o, 0)),
        ],
        out_specs=pl.BlockSpec((BATCH, bo), lambda o: (0, o)),
        out_shape=jax.ShapeDtypeStruct((BATCH, OUT_FEATURES), jnp.float32),
    )(x, wd)


def kernel(input, connections, weights):
    cw = jnp.concatenate(
        [connections, lax.bitcast_convert_type(weights, jnp.int32)], axis=1)
    x_bf = input.astype(jnp.bfloat16)
    wd = _densify_sc(cw)
    return _matmul_tc(x_bf, wd)
